# baseline (speedup 1.0000x reference)
import functools

import jax
import jax.numpy as jnp
from jax.experimental import pallas as pl
from jax.experimental.pallas import tpu as pltpu

N = 32768
B = 16
D = 1024
EPS = 1e-05

ROWS = 1024


def _onehot(starts_ref, ends_ref, step, rows):
    r = step * rows + jax.lax.broadcasted_iota(jnp.int32, (rows, B), 0)
    return ((r >= starts_ref[...]) & (r < ends_ref[...])).astype(jnp.float32)


def _fused_kernel(x_ref, starts_ref, ends_ref, w_ref, b_ref, invc_ref,
                  out_ref, sum_s, sq_s, scale_s, shift_s):
    p = pl.program_id(0)
    i = pl.program_id(1)
    oh = _onehot(starts_ref, ends_ref, i, ROWS)

    @pl.when(p == 0)
    def _():
        x = x_ref[...]
        dims = (((0,), (0,)), ((), ()))
        ps = jax.lax.dot_general(oh, x, dims,
                                 preferred_element_type=jnp.float32)
        psq = jax.lax.dot_general(oh, x * x, dims,
                                  preferred_element_type=jnp.float32)

        @pl.when(i == 0)
        def _():
            sum_s[...] = ps
            sq_s[...] = psq

        @pl.when(i != 0)
        def _():
            sum_s[...] += ps
            sq_s[...] += psq

    @pl.when((p == 1) & (i == 0))
    def _():
        inv = invc_ref[:, 0:1]
        mean = sum_s[...] * inv
        var = sq_s[...] * inv - mean * mean
        rstd = jax.lax.rsqrt(jnp.maximum(var, 0.0) + EPS)
        scale = rstd * w_ref[...]
        scale_s[...] = scale
        shift_s[...] = b_ref[...] - mean * scale

    @pl.when(p == 1)
    def _():
        dims = (((1,), (0,)), ((), ()))
        row_scale = jax.lax.dot_general(oh, scale_s[...], dims,
                                        preferred_element_type=jnp.float32)
        row_shift = jax.lax.dot_general(oh, shift_s[...], dims,
                                        preferred_element_type=jnp.float32)
        out_ref[...] = x_ref[...] * row_scale + row_shift


@functools.partial(jax.jit, static_argnames=("interpret",))
def kernel(input, offsets, weight, bias, interpret=False):
    steps = N // ROWS
    ends = offsets.reshape(1, B)
    starts = jnp.concatenate(
        [jnp.zeros((1, 1), jnp.int32), ends[:, :-1]], axis=1)
    invc = jnp.broadcast_to(
        (1.0 / jnp.maximum(ends - starts, 1).astype(jnp.float32)).reshape(
            B, 1), (B, 128))

    small = pl.BlockSpec((1, B), lambda p, i: (0, 0))
    out = pl.pallas_call(
        _fused_kernel,
        grid=(2, steps),
        in_specs=[pl.BlockSpec((ROWS, D), lambda p, i: (i, 0)),
                  small, small,
                  pl.BlockSpec((1, D), lambda p, i: (0, 0)),
                  pl.BlockSpec((1, D), lambda p, i: (0, 0)),
                  pl.BlockSpec((B, 128), lambda p, i: (0, 0))],
        out_specs=pl.BlockSpec((ROWS, D), lambda p, i: (i * p, 0)),
        out_shape=jax.ShapeDtypeStruct((N, D), jnp.float32),
        scratch_shapes=[pltpu.VMEM((B, D), jnp.float32),
                        pltpu.VMEM((B, D), jnp.float32),
                        pltpu.VMEM((B, D), jnp.float32),
                        pltpu.VMEM((B, D), jnp.float32)],
        interpret=interpret,
    )(input, starts, ends, weight.reshape(1, D), bias.reshape(1, D), invc)
    return out


# speedup vs baseline: 1.0925x; 1.0925x over previous
import functools

import jax
import jax.numpy as jnp
from jax.experimental import pallas as pl
from jax.experimental.pallas import tpu as pltpu

N = 32768
B = 16
D = 1024
EPS = 1e-05

ROWS = 2048


def _onehot(starts_ref, ends_ref, step, rows):
    r = step * rows + jax.lax.broadcasted_iota(jnp.int32, (rows, B), 0)
    return ((r >= starts_ref[...]) & (r < ends_ref[...])).astype(jnp.float32)


def _fused_kernel(x_ref, starts_ref, ends_ref, w_ref, b_ref, invc_ref,
                  out_ref, sum_s, sq_s, scale_s, shift_s):
    p = pl.program_id(0)
    i = pl.program_id(1)
    oh = _onehot(starts_ref, ends_ref, i, ROWS)

    @pl.when(p == 0)
    def _():
        x = x_ref[...]
        dims = (((0,), (0,)), ((), ()))
        ps = jax.lax.dot_general(oh, x, dims,
                                 preferred_element_type=jnp.float32)
        psq = jax.lax.dot_general(oh, x * x, dims,
                                  preferred_element_type=jnp.float32)

        @pl.when(i == 0)
        def _():
            sum_s[...] = ps
            sq_s[...] = psq

        @pl.when(i != 0)
        def _():
            sum_s[...] += ps
            sq_s[...] += psq

    @pl.when((p == 1) & (i == 0))
    def _():
        inv = invc_ref[:, 0:1]
        mean = sum_s[...] * inv
        var = sq_s[...] * inv - mean * mean
        rstd = jax.lax.rsqrt(jnp.maximum(var, 0.0) + EPS)
        scale = rstd * w_ref[...]
        scale_s[...] = scale
        shift_s[...] = b_ref[...] - mean * scale

    @pl.when(p == 1)
    def _():
        dims = (((1,), (0,)), ((), ()))
        row_scale = jax.lax.dot_general(oh, scale_s[...], dims,
                                        preferred_element_type=jnp.float32)
        row_shift = jax.lax.dot_general(oh, shift_s[...], dims,
                                        preferred_element_type=jnp.float32)
        out_ref[...] = x_ref[...] * row_scale + row_shift


@functools.partial(jax.jit, static_argnames=("interpret",))
def kernel(input, offsets, weight, bias, interpret=False):
    steps = N // ROWS
    ends = offsets.reshape(1, B)
    starts = jnp.concatenate(
        [jnp.zeros((1, 1), jnp.int32), ends[:, :-1]], axis=1)
    invc = jnp.broadcast_to(
        (1.0 / jnp.maximum(ends - starts, 1).astype(jnp.float32)).reshape(
            B, 1), (B, 128))

    small = pl.BlockSpec((1, B), lambda p, i: (0, 0))
    out = pl.pallas_call(
        _fused_kernel,
        grid=(2, steps),
        in_specs=[pl.BlockSpec((ROWS, D), lambda p, i: (i, 0)),
                  small, small,
                  pl.BlockSpec((1, D), lambda p, i: (0, 0)),
                  pl.BlockSpec((1, D), lambda p, i: (0, 0)),
                  pl.BlockSpec((B, 128), lambda p, i: (0, 0))],
        out_specs=pl.BlockSpec((ROWS, D), lambda p, i: (i * p, 0)),
        out_shape=jax.ShapeDtypeStruct((N, D), jnp.float32),
        scratch_shapes=[pltpu.VMEM((B, D), jnp.float32),
                        pltpu.VMEM((B, D), jnp.float32),
                        pltpu.VMEM((B, D), jnp.float32),
                        pltpu.VMEM((B, D), jnp.float32)],
        interpret=interpret,
    )(input, starts, ends, weight.reshape(1, D), bias.reshape(1, D), invc)
    return out
